# baseline (speedup 1.0000x reference)
import functools

import jax
import jax.numpy as jnp
from jax import lax
from jax.experimental import pallas as pl
from jax.experimental.pallas import tpu as pltpu
import jax.experimental.pallas.tpu_sc as plsc

B, T, C, Q, V, D = 1024, 50, 2, 8, 8192, 64
NC, NS = 2, 16
NW = NC * NS
N = B * T * C * Q
RPW = N // NW
CH = 256
NCHUNK = RPW // CH
GSUB = 128
SUB = CH // GSUB
NG = CH // 16


def _expand_body(tab_ref, ch_ref, out_ref):
    q = pl.program_id(0)
    t = tab_ref[0]
    b0 = ch_ref[pl.ds(q, 1), :]
    b1 = ch_ref[pl.ds(Q + q, 1), :]
    out_ref[...] = jnp.concatenate([t + b0, t + b1], axis=1)


def _expand_table(tables, channel_emb):
    return pl.pallas_call(
        _expand_body,
        grid=(Q,),
        in_specs=[
            pl.BlockSpec((1, V, D), lambda q: (q, 0, 0)),
            pl.BlockSpec((C * Q, D), lambda q: (0, 0)),
        ],
        out_specs=pl.BlockSpec((V, 2 * D), lambda q: (q, 0)),
        out_shape=jax.ShapeDtypeStruct((Q * V, 2 * D), jnp.float32),
    )(tables, channel_emb.reshape(C * Q, D))


def _gather_body(codes_hbm, exp_hbm, out_hbm, idx_a, idx_b, rows_a, rows_b,
                 gsem_a, gsem_b, wsem_a, wsem_b):
    wid = lax.axis_index("s") * NC + lax.axis_index("c")
    base = wid * RPW
    lane = lax.iota(jnp.int32, 16)
    offs = (lane & (Q - 1)) << 13

    def fire(g, idx_v, rows_v, gsem):
        row0 = pl.multiple_of(base + g * CH, CH)
        pltpu.sync_copy(
            codes_hbm.at[pl.ds(pl.multiple_of(row0 // GSUB, SUB), SUB)], idx_v
        )
        for i in range(SUB):
            for j in range(GSUB // 16):
                sl = pl.ds(j * 16, 16)
                idx_v[i, sl] = idx_v[i, sl] + offs
        for i in range(SUB):
            pltpu.async_copy(
                exp_hbm.at[idx_v.at[i]], rows_v.at[pl.ds(i * GSUB, GSUB)], gsem
            )

    def drain_gathers(idx_v, rows_v, gsem):
        for i in range(SUB):
            pltpu.make_async_copy(
                exp_hbm.at[idx_v.at[i]], rows_v.at[pl.ds(i * GSUB, GSUB)], gsem
            ).wait()

    def write_descs(g, rows_v, wsem):
        row0 = pl.multiple_of(base + g * CH, CH)
        for k in range(NG):
            for h in range(2):
                yield (
                    rows_v.at[pl.ds(k * 16 + h * 8, 8), pl.ds(h * D, D)],
                    out_hbm.at[pl.ds(row0 + k * 16 + h * 8, 8)],
                    wsem,
                )

    def write(g, rows_v, wsem):
        for src, dst, sem in write_descs(g, rows_v, wsem):
            pltpu.async_copy(src, dst, sem)

    def drain_writes(g, rows_v, wsem):
        for src, dst, sem in write_descs(g, rows_v, wsem):
            pltpu.make_async_copy(src, dst, sem).wait()

    fire(0, idx_a, rows_a, gsem_a)

    @pl.loop(0, NCHUNK, step=2)
    def _pair(g):
        @pl.when(g > 0)
        def _():
            drain_writes(g - 1, rows_b, wsem_b)

        fire(g + 1, idx_b, rows_b, gsem_b)
        drain_gathers(idx_a, rows_a, gsem_a)
        write(g, rows_a, wsem_a)

        @pl.when(g + 2 < NCHUNK)
        def _():
            drain_writes(g, rows_a, wsem_a)
            fire(g + 2, idx_a, rows_a, gsem_a)

        drain_gathers(idx_b, rows_b, gsem_b)
        write(g + 1, rows_b, wsem_b)

    drain_writes(NCHUNK - 2, rows_a, wsem_a)
    drain_writes(NCHUNK - 1, rows_b, wsem_b)


@functools.cache
def _make_gather():
    return pl.kernel(
        _gather_body,
        out_type=jax.ShapeDtypeStruct((N, D), jnp.float32),
        mesh=plsc.VectorSubcoreMesh(
            core_axis_name="c", subcore_axis_name="s", num_cores=NC, num_subcores=NS
        ),
        scratch_types=[
            pltpu.VMEM((SUB, GSUB), jnp.int32),
            pltpu.VMEM((SUB, GSUB), jnp.int32),
            pltpu.VMEM((CH, 2 * D), jnp.float32),
            pltpu.VMEM((CH, 2 * D), jnp.float32),
            pltpu.SemaphoreType.DMA,
            pltpu.SemaphoreType.DMA,
            pltpu.SemaphoreType.DMA,
            pltpu.SemaphoreType.DMA,
        ],
        compiler_params=pltpu.CompilerParams(use_tc_tiling_on_sc=False),
    )


def kernel(codes, tables, channel_emb):
    exp = _expand_table(tables, channel_emb)
    _gather = _make_gather()
    codes2 = codes.astype(jnp.int32).reshape(N // GSUB, GSUB)
    out = _gather(codes2, exp)
    return out.reshape(B, T, C * Q * D)
